# Initial kernel scaffold; baseline (speedup 1.0000x reference)
import jax
import jax.numpy as jnp
from jax.experimental import pallas as pl


def kernel(action, table):
    raise NotImplementedError("write your pallas kernel here")


# speedup vs baseline: 2.4475x; 2.4475x over previous
import functools

import jax
import jax.numpy as jnp
from jax import lax
from jax.experimental import pallas as pl
from jax.experimental.pallas import tpu as pltpu
from jax.experimental.pallas import tpu_sc as plsc

VOCAB = 1000000
DIM = 16
B, L = 16384, 200
N = B * L
NC, NS, LANES = 2, 16, 16
NW = NC * NS
NPW = N // NW
CHUNK = 2048
NCHUNK = NPW // CHUNK


def _body(idx_hbm, table_hbm, out_hbm, idx_v, rows_v, gsem):
    wid = lax.axis_index("s") * NC + lax.axis_index("c")
    base = wid * NPW

    def chunk_step(c, carry):
        off = base + c * CHUNK
        pltpu.sync_copy(idx_hbm.at[pl.ds(off, CHUNK)], idx_v)

        def add_one(i, carry2):
            sl = pl.ds(i * LANES, LANES)
            idx_v[sl] = idx_v[sl] + 1
            return carry2

        lax.fori_loop(0, CHUNK // LANES, add_one, 0, unroll=8)
        pltpu.async_copy(table_hbm.at[idx_v], rows_v, gsem).wait()
        pltpu.sync_copy(rows_v, out_hbm.at[pl.ds(off, CHUNK)])
        return carry

    lax.fori_loop(0, NCHUNK, chunk_step, 0)


def kernel(action, table):
    idx = action.reshape(N).astype(jnp.int32)
    mesh = plsc.VectorSubcoreMesh(
        core_axis_name="c", subcore_axis_name="s", num_cores=NC,
        num_subcores=NS)
    out = pl.kernel(
        _body,
        out_type=jax.ShapeDtypeStruct((N, DIM), jnp.float32),
        mesh=mesh,
        scratch_types=[
            pltpu.VMEM((CHUNK,), jnp.int32),
            pltpu.VMEM((CHUNK, DIM), jnp.float32),
            pltpu.SemaphoreType.DMA,
        ],
        compiler_params=pltpu.CompilerParams(use_tc_tiling_on_sc=False),
    )(idx, table)
    return out.reshape(B, L, DIM)


# speedup vs baseline: 2.4952x; 1.0195x over previous
import functools

import jax
import jax.numpy as jnp
from jax import lax
from jax.experimental import pallas as pl
from jax.experimental.pallas import tpu as pltpu
from jax.experimental.pallas import tpu_sc as plsc

VOCAB = 1000000
DIM = 16
B, L = 16384, 200
N = B * L
NC, NS, LANES = 2, 16, 16
NW = NC * NS
NPW = N // NW
CHUNK = 2048
NCHUNK = NPW // CHUNK


def _body(idx_hbm, table_hbm, out_hbm, idx_v, rows_v,
          gsem0, gsem1, wsem0, wsem1):
    wid = lax.axis_index("s") * NC + lax.axis_index("c")
    base = wid * NPW
    gsems = (gsem0, gsem1)
    wsems = (wsem0, wsem1)

    def load_add(c, b):
        off = base + c * CHUNK
        pltpu.sync_copy(idx_hbm.at[pl.ds(off, CHUNK)], idx_v.at[b])

        def add_one(i, carry):
            sl = pl.ds(i * LANES, LANES)
            idx_v[b, sl] = idx_v[b, sl] + 1
            return carry

        lax.fori_loop(0, CHUNK // LANES, add_one, 0, unroll=8)

    def start_gather(b):
        pltpu.async_copy(table_hbm.at[idx_v.at[b]], rows_v.at[b], gsems[b])

    def wait_gather(b):
        pltpu.make_async_copy(table_hbm.at[idx_v.at[b]], rows_v.at[b],
                              gsems[b]).wait()

    def start_write(c, b):
        off = base + c * CHUNK
        pltpu.async_copy(rows_v.at[b], out_hbm.at[pl.ds(off, CHUNK)],
                         wsems[b])

    def wait_write(c, b):
        off = base + c * CHUNK
        pltpu.make_async_copy(rows_v.at[b], out_hbm.at[pl.ds(off, CHUNK)],
                              wsems[b]).wait()

    load_add(0, 0)
    start_gather(0)

    def outer(t, carry):
        for b in (0, 1):
            c = t * 2 + b
            bnext = 1 - b

            @pl.when(c + 1 < NCHUNK)
            def _():
                load_add(c + 1, bnext)

            @pl.when(c >= 1)
            def _():
                wait_write(c - 1, bnext)

            wait_gather(b)

            @pl.when(c + 1 < NCHUNK)
            def _():
                start_gather(bnext)

            start_write(c, b)
        return carry

    lax.fori_loop(0, NCHUNK // 2, outer, 0)
    wait_write(NCHUNK - 1, 1)


def kernel(action, table):
    idx = action.reshape(N).astype(jnp.int32)
    mesh = plsc.VectorSubcoreMesh(
        core_axis_name="c", subcore_axis_name="s", num_cores=NC,
        num_subcores=NS)
    out = pl.kernel(
        _body,
        out_type=jax.ShapeDtypeStruct((N, DIM), jnp.float32),
        mesh=mesh,
        scratch_types=[
            pltpu.VMEM((2, CHUNK), jnp.int32),
            pltpu.VMEM((2, CHUNK, DIM), jnp.float32),
            pltpu.SemaphoreType.DMA,
            pltpu.SemaphoreType.DMA,
            pltpu.SemaphoreType.DMA,
            pltpu.SemaphoreType.DMA,
        ],
        compiler_params=pltpu.CompilerParams(use_tc_tiling_on_sc=False),
    )(idx, table)
    return out.reshape(B, L, DIM)


# speedup vs baseline: 4.6625x; 1.8685x over previous
import jax
import jax.numpy as jnp
from jax import lax
from jax.experimental import pallas as pl
from jax.experimental.pallas import tpu as pltpu
from jax.experimental.pallas import tpu_sc as plsc

VOCAB = 1000000
DIM = 16
B, L = 16384, 200
N = B * L
NC, NS, LANES = 2, 16, 16
NW = NC * NS
SB = 1024
NSB = B // SB
UNITS = L * NSB
UPT = UNITS // NW
BT = B // 128


def _body(idx_hbm, table_hbm, out_hbm, idxb, rows, stag, gsem, wsem0, wsem1):
    wid = lax.axis_index("s") * NC + lax.axis_index("c")
    u0 = wid * UPT
    lane = lax.iota(jnp.int32, LANES)
    wsems = (wsem0, wsem1)

    def unit_lsb(t):
        u = u0 + t
        return u >> 4, u & (NSB - 1)

    def load_add(t, buf):
        l, sb = unit_lsb(t)
        off = l * B + sb * SB
        pltpu.sync_copy(idx_hbm.at[pl.ds(off, SB)], idxb.at[buf])

        def add_one(i, carry):
            sl = pl.ds(i * LANES, LANES)
            idxb[buf, sl] = idxb[buf, sl] + 1
            return carry

        lax.fori_loop(0, SB // LANES, add_one, 0, unroll=8)

    def start_gather(buf):
        pltpu.async_copy(table_hbm.at[idxb.at[buf]], rows.at[buf], gsem)

    def wait_gather(buf):
        pltpu.make_async_copy(table_hbm.at[idxb.at[buf]], rows.at[buf],
                              gsem).wait()

    def transpose(buf):
        rbuf = rows.at[buf]
        sbuf = stag.at[buf]

        def step(it, carry):
            dt = it >> 9
            bt8 = (it >> 6) & 7
            dsub = (it >> 3) & 7
            j = it & 7
            row_vec = lane + (bt8 * 128 + j * LANES)
            col = dt * 8 + dsub
            col_vec = jnp.full((LANES,), 0, jnp.int32) + col
            v = plsc.load_gather(rbuf, [row_vec, col_vec])
            sbuf[dt, bt8, dsub, pl.ds(j * LANES, LANES)] = v
            return carry

        lax.fori_loop(0, SB * DIM // LANES, step, 0, unroll=8)

    def start_write(t, buf):
        l, sb = unit_lsb(t)
        for dt in (0, 1):
            pltpu.async_copy(stag.at[buf, dt],
                             out_hbm.at[l, dt, pl.ds(sb * 8, 8)], wsems[dt])

    def wait_write(t, buf):
        l, sb = unit_lsb(t)
        for dt in (0, 1):
            pltpu.make_async_copy(stag.at[buf, dt],
                                  out_hbm.at[l, dt, pl.ds(sb * 8, 8)],
                                  wsems[dt]).wait()

    load_add(0, 0)
    start_gather(0)

    def outer(tt, carry):
        for b in (0, 1):
            t = tt * 2 + b
            bnext = 1 - b

            @pl.when(t + 1 < UPT)
            def _():
                load_add(t + 1, bnext)

            wait_gather(b)

            @pl.when(t + 1 < UPT)
            def _():
                start_gather(bnext)

            @pl.when(t >= 2)
            def _():
                wait_write(t - 2, b)

            transpose(b)
            start_write(t, b)
        return carry

    lax.fori_loop(0, UPT // 2, outer, 0)
    wait_write(UPT - 2, 0)
    wait_write(UPT - 1, 1)


def kernel(action, table):
    idx1d = action.reshape(B, L).T.reshape(N)
    mesh = plsc.VectorSubcoreMesh(
        core_axis_name="c", subcore_axis_name="s", num_cores=NC,
        num_subcores=NS)
    out6 = pl.kernel(
        _body,
        out_type=jax.ShapeDtypeStruct((L, 2, BT, 8, 128), jnp.float32),
        mesh=mesh,
        scratch_types=[
            pltpu.VMEM((2, SB), jnp.int32),
            pltpu.VMEM((2, SB, DIM), jnp.float32),
            pltpu.VMEM((2, 2, 8, 8, 128), jnp.float32),
            pltpu.SemaphoreType.DMA,
            pltpu.SemaphoreType.DMA,
            pltpu.SemaphoreType.DMA,
        ],
        compiler_params=pltpu.CompilerParams(use_tc_tiling_on_sc=False,
                                             needs_layout_passes=False),
    )(idx1d, table)
    return out6.transpose(2, 4, 0, 1, 3).reshape(B, L, DIM)
